# Initial kernel scaffold; baseline (speedup 1.0000x reference)
import jax
import jax.numpy as jnp
from jax.experimental import pallas as pl


def kernel(x, W):
    raise NotImplementedError("write your pallas kernel here")


# speedup vs baseline: 105.9757x; 105.9757x over previous
import functools

import jax
import jax.numpy as jnp
from jax import lax
from jax.experimental import pallas as pl
from jax.experimental.pallas import tpu as pltpu
from jax.experimental.pallas import tpu_sc as plsc

_N_BINS = 256
_EMB = 32
_NC = 2
_NS = 16
_NW = _NC * _NS
_LANES = 16

_CHUNK = 1024
_GATHER = 128
_G_PER_CHUNK = _CHUNK // _GATHER

_BINS_PAD = 272


def _tanh_table_body(w_ref, o_ref):
    o_ref[...] = jnp.tanh(w_ref[...])


def _sc_body(x_hbm, tw_hbm, out_hbm, xv, idxv, rowsv, gsem):
    n_total = x_hbm.shape[0]
    per_w = n_total // _NW
    n_chunks = per_w // _CHUNK

    wid = lax.axis_index("s") * _NC + lax.axis_index("c")
    base = wid * per_w

    step = jnp.float32(1.0) / jnp.float32(254.0)

    def chunk_body(c, carry):
        off = base + c * _CHUNK
        pltpu.sync_copy(x_hbm.at[pl.ds(off, _CHUNK)], xv)

        for i in range(_CHUNK // _LANES):
            xx = xv[pl.ds(i * _LANES, _LANES)]
            t = xx * 254.0 + 0.5
            i0 = t.astype(jnp.int32)
            i0 = jnp.minimum(jnp.maximum(i0, 0), 255)
            f = i0.astype(jnp.float32)
            hi = f * step
            lo = (f - 1.0) * step
            idx = (i0
                   + jnp.where(xx > hi, 1, 0)
                   - jnp.where(xx <= lo, 1, 0))
            idxv[i // (_GATHER // _LANES),
                 pl.ds((i % (_GATHER // _LANES)) * _LANES, _LANES)] = idx

        copies = []
        for j in range(_G_PER_CHUNK):
            copies.append(
                pltpu.async_copy(
                    tw_hbm.at[idxv.at[j]],
                    rowsv.at[pl.ds(j * _GATHER, _GATHER)],
                    gsem,
                ))
        for cp in copies:
            cp.wait()

        pltpu.sync_copy(rowsv, out_hbm.at[pl.ds(off, _CHUNK)])
        return carry

    lax.fori_loop(0, n_chunks, chunk_body, 0)


def kernel(x, W):
    bsz, tsz = x.shape
    n_total = bsz * tsz
    x_flat = x.reshape(n_total)

    tw = pl.pallas_call(
        _tanh_table_body,
        out_shape=jax.ShapeDtypeStruct((_N_BINS, _EMB), jnp.float32),
    )(W)

    mesh = plsc.VectorSubcoreMesh(
        core_axis_name="c", subcore_axis_name="s",
        num_cores=_NC, num_subcores=_NS)

    sc = functools.partial(
        pl.kernel,
        mesh=mesh,
        out_type=jax.ShapeDtypeStruct((n_total, _EMB), jnp.float32),
        scratch_types=[
            pltpu.VMEM((_CHUNK,), jnp.float32),
            pltpu.VMEM((_G_PER_CHUNK, _GATHER), jnp.int32),
            pltpu.VMEM((_CHUNK, _EMB), jnp.float32),
            pltpu.SemaphoreType.DMA,
        ],
        compiler_params=pltpu.CompilerParams(use_tc_tiling_on_sc=False),
    )(_sc_body)

    out = sc(x_flat, tw)
    return out.reshape(bsz, tsz, _EMB)


# speedup vs baseline: 106.3782x; 1.0038x over previous
import functools

import jax
import jax.numpy as jnp
from jax import lax
from jax.experimental import pallas as pl
from jax.experimental.pallas import tpu as pltpu
from jax.experimental.pallas import tpu_sc as plsc

_N_BINS = 256
_EMB = 32
_NC = 2
_NS = 16
_NW = _NC * _NS
_LANES = 16

_CHUNK = 1024
_GATHER = 128
_G_PER_CHUNK = _CHUNK // _GATHER

_BINS_PAD = 272


def _tanh_table_body(w_ref, o_ref):
    o_ref[...] = jnp.tanh(w_ref[...])


def _sc_body(x_hbm, tw_hbm, out_hbm,
             xv0, xv1, idxv0, idxv1, rows0, rows1, xsem, gsem, osem):
    n_total = x_hbm.shape[0]
    per_w = n_total // _NW
    n_chunks = per_w // _CHUNK

    wid = lax.axis_index("s") * _NC + lax.axis_index("c")
    base = wid * per_w

    step = jnp.float32(1.0) / jnp.float32(254.0)
    xvs, idxvs, rowss = (xv0, xv1), (idxv0, idxv1), (rows0, rows1)

    def off(c):
        return base + c * _CHUNK

    pltpu.async_copy(x_hbm.at[pl.ds(off(0), _CHUNK)], xv0, xsem)

    def one_iter(c, b):
        xv, idxv, rowsv = xvs[b], idxvs[b], rowss[b]
        pltpu.make_async_copy(
            x_hbm.at[pl.ds(off(c), _CHUNK)], xv, xsem).wait()

        @pl.when(c <= n_chunks - 2)
        def _():
            pltpu.async_copy(
                x_hbm.at[pl.ds(off(c + 1), _CHUNK)], xvs[1 - b], xsem)

        for i in range(_CHUNK // _LANES):
            xx = xv[pl.ds(i * _LANES, _LANES)]
            t = xx * 254.0 + 0.5
            i0 = t.astype(jnp.int32)
            i0 = jnp.minimum(jnp.maximum(i0, 0), 255)
            f = i0.astype(jnp.float32)
            hi = f * step
            lo = (f - 1.0) * step
            idx = (i0
                   + jnp.where(xx > hi, 1, 0)
                   - jnp.where(xx <= lo, 1, 0))
            idxv[i // (_GATHER // _LANES),
                 pl.ds((i % (_GATHER // _LANES)) * _LANES, _LANES)] = idx

        @pl.when(c >= 2)
        def _():
            pltpu.make_async_copy(
                rowsv, out_hbm.at[pl.ds(off(c - 2), _CHUNK)], osem).wait()

        copies = []
        for j in range(_G_PER_CHUNK):
            copies.append(
                pltpu.async_copy(
                    tw_hbm.at[idxv.at[j]],
                    rowsv.at[pl.ds(j * _GATHER, _GATHER)],
                    gsem,
                ))
        for cp in copies:
            cp.wait()

        pltpu.async_copy(rowsv, out_hbm.at[pl.ds(off(c), _CHUNK)], osem)

    def super_body(s, carry):
        one_iter(2 * s, 0)
        one_iter(2 * s + 1, 1)
        return carry

    lax.fori_loop(0, n_chunks // 2, super_body, 0)

    pltpu.make_async_copy(
        rows0, out_hbm.at[pl.ds(off(n_chunks - 2), _CHUNK)], osem).wait()
    pltpu.make_async_copy(
        rows1, out_hbm.at[pl.ds(off(n_chunks - 1), _CHUNK)], osem).wait()


def kernel(x, W):
    bsz, tsz = x.shape
    n_total = bsz * tsz
    x_flat = x.reshape(n_total)

    tw = pl.pallas_call(
        _tanh_table_body,
        out_shape=jax.ShapeDtypeStruct((_N_BINS, _EMB), jnp.float32),
    )(W)

    mesh = plsc.VectorSubcoreMesh(
        core_axis_name="c", subcore_axis_name="s",
        num_cores=_NC, num_subcores=_NS)

    sc = functools.partial(
        pl.kernel,
        mesh=mesh,
        out_type=jax.ShapeDtypeStruct((n_total, _EMB), jnp.float32),
        scratch_types=[
            pltpu.VMEM((_CHUNK,), jnp.float32),
            pltpu.VMEM((_CHUNK,), jnp.float32),
            pltpu.VMEM((_G_PER_CHUNK, _GATHER), jnp.int32),
            pltpu.VMEM((_G_PER_CHUNK, _GATHER), jnp.int32),
            pltpu.VMEM((_CHUNK, _EMB), jnp.float32),
            pltpu.VMEM((_CHUNK, _EMB), jnp.float32),
            pltpu.SemaphoreType.DMA,
            pltpu.SemaphoreType.DMA,
            pltpu.SemaphoreType.DMA,
        ],
        compiler_params=pltpu.CompilerParams(use_tc_tiling_on_sc=False),
    )(_sc_body)

    out = sc(x_flat, tw)
    return out.reshape(bsz, tsz, _EMB)


# speedup vs baseline: 109.3407x; 1.0278x over previous
import functools

import jax
import jax.numpy as jnp
from jax import lax
from jax.experimental import pallas as pl
from jax.experimental.pallas import tpu as pltpu
from jax.experimental.pallas import tpu_sc as plsc

_N_BINS = 256
_EMB = 32
_NC = 2
_NS = 16
_NW = _NC * _NS
_LANES = 16

_TBLK = 8
_BPW = 512
_KV = _BPW // _LANES


def _tanh_table_body(w_ref, o_ref):
    o_ref[...] = jnp.tanh(w_ref[...])


def _sc_body(xt_hbm, tw_hbm, out_hbm, tabv, xblk, stg0, stg1, xsem, osem):
    n_t = xt_hbm.shape[0]

    wid = lax.axis_index("s") * _NC + lax.axis_index("c")
    b0 = wid * _BPW
    bt0 = wid * (_BPW // 128)

    step = jnp.float32(1.0) / jnp.float32(254.0)
    stgs = (stg0, stg1)

    pltpu.sync_copy(tw_hbm, tabv)

    dconsts = [jnp.full((_LANES,), d, jnp.int32) for d in range(_EMB)]

    def gather_t(tl, t, stg):
        def k_body(k, carry):
            xx = xblk[tl, pl.ds(k * _LANES, _LANES)]
            tt = xx * 254.0 + 0.5
            i0 = tt.astype(jnp.int32)
            i0 = jnp.minimum(jnp.maximum(i0, 0), 255)
            f = i0.astype(jnp.float32)
            hi = f * step
            lo = (f - 1.0) * step
            idx = (i0
                   + jnp.where(xx > hi, 1, 0)
                   - jnp.where(xx <= lo, 1, 0))
            kb = k // 8
            bl = (k % 8) * _LANES
            for d in range(_EMB):
                g = plsc.load_gather(tabv, [idx, dconsts[d]])
                stg[d // 8, kb, d % 8, pl.ds(bl, _LANES)] = g
            return carry

        lax.fori_loop(0, _KV, k_body, 0)

    def blk_body(s, carry):
        t0 = s * _TBLK
        pltpu.sync_copy(
            xt_hbm.at[pl.ds(t0, _TBLK), pl.ds(b0, _BPW)], xblk)

        for tl in range(_TBLK):
            t = t0 + tl
            stg = stgs[tl % 2]

            if tl >= 2:
                pltpu.make_async_copy(
                    stg, out_hbm.at[t - 2, :, pl.ds(bt0, _BPW // 128)],
                    osem).wait()
            else:
                @pl.when(s > 0)
                def _():
                    pltpu.make_async_copy(
                        stg, out_hbm.at[t - 2, :, pl.ds(bt0, _BPW // 128)],
                        osem).wait()

            gather_t(tl, t, stg)
            pltpu.async_copy(
                stg, out_hbm.at[t, :, pl.ds(bt0, _BPW // 128)], osem)
        return carry

    lax.fori_loop(0, n_t // _TBLK, blk_body, 0)

    for t in (n_t - 2, n_t - 1):
        pltpu.make_async_copy(
            stgs[t % 2], out_hbm.at[t, :, pl.ds(bt0, _BPW // 128)],
            osem).wait()


def kernel(x, W):
    bsz, tsz = x.shape
    n_emb, emb = W.shape

    tw = pl.pallas_call(
        _tanh_table_body,
        out_shape=jax.ShapeDtypeStruct((n_emb, emb), jnp.float32),
    )(W)

    xt = jnp.transpose(x)

    mesh = plsc.VectorSubcoreMesh(
        core_axis_name="c", subcore_axis_name="s",
        num_cores=_NC, num_subcores=_NS)

    sc = functools.partial(
        pl.kernel,
        mesh=mesh,
        out_type=jax.ShapeDtypeStruct(
            (tsz, emb // 8, bsz // 128, 8, 128), jnp.float32),
        scratch_types=[
            pltpu.VMEM((n_emb, emb), jnp.float32),
            pltpu.VMEM((_TBLK, _BPW), jnp.float32),
            pltpu.VMEM((emb // 8, _BPW // 128, 8, 128), jnp.float32),
            pltpu.VMEM((emb // 8, _BPW // 128, 8, 128), jnp.float32),
            pltpu.SemaphoreType.DMA,
            pltpu.SemaphoreType.DMA,
        ],
        compiler_params=pltpu.CompilerParams(
            use_tc_tiling_on_sc=False, needs_layout_passes=False),
    )(_sc_body)

    out5 = sc(xt, tw)
    out = out5.transpose(2, 4, 0, 1, 3).reshape(bsz, tsz, emb)
    return out


# speedup vs baseline: 274.7289x; 2.5126x over previous
import functools

import jax
import jax.numpy as jnp
from jax import lax
from jax.experimental import pallas as pl
from jax.experimental.pallas import tpu as pltpu
from jax.experimental.pallas import tpu_sc as plsc

_N_BINS = 256
_EMB = 32
_NC = 2
_NS = 16
_NW = _NC * _NS
_LANES = 16

_TBLK = 8
_BPW = 512
_KV = _BPW // _LANES
_STRIDE = 33


def _tanh_table_body(w_ref, o_ref):
    o_ref[...] = jnp.tanh(w_ref[...])


def _sc_body(xt_hbm, tw_hbm, out_hbm, tabv, tab33, xblk, stg0, stg1,
             xsem, osem):
    n_t = xt_hbm.shape[0]

    wid = lax.axis_index("s") * _NC + lax.axis_index("c")
    b0 = wid * _BPW
    bt0 = wid * (_BPW // 128)

    step = jnp.float32(1.0) / jnp.float32(254.0)
    stgs = (stg0, stg1)

    pltpu.sync_copy(tw_hbm, tabv)

    def restride(r, carry):
        tab33[pl.ds(r * _STRIDE, _LANES)] = tabv[r, pl.ds(0, _LANES)]
        tab33[pl.ds(r * _STRIDE + _LANES, _LANES)] = tabv[r, pl.ds(_LANES, _LANES)]
        return carry

    lax.fori_loop(0, _N_BINS, restride, 0)

    def gather_t(tl, stg):
        def k_body(k, carry):
            xx = xblk[tl, pl.ds(k * _LANES, _LANES)]
            tt = xx * 254.0 + 0.5
            i0 = tt.astype(jnp.int32)
            i0 = jnp.minimum(jnp.maximum(i0, 0), 255)
            f = i0.astype(jnp.float32)
            hi = f * step
            lo = (f - 1.0) * step
            idx = (i0
                   + jnp.where(xx > hi, 1, 0)
                   - jnp.where(xx <= lo, 1, 0))
            idx33 = idx * _STRIDE
            kb = k // 8
            bl = (k % 8) * _LANES
            for d in range(_EMB):
                g = plsc.load_gather(tab33, [idx33 + d])
                stg[d // 8, kb, d % 8, pl.ds(bl, _LANES)] = g
            return carry

        lax.fori_loop(0, _KV, k_body, 0)

    def blk_body(s, carry):
        t0 = s * _TBLK
        pltpu.sync_copy(
            xt_hbm.at[pl.ds(t0, _TBLK), pl.ds(b0, _BPW)], xblk)

        for tl in range(_TBLK):
            t = t0 + tl
            stg = stgs[tl % 2]

            if tl >= 2:
                pltpu.make_async_copy(
                    stg, out_hbm.at[t - 2, :, pl.ds(bt0, _BPW // 128)],
                    osem).wait()
            else:
                @pl.when(s > 0)
                def _():
                    pltpu.make_async_copy(
                        stg, out_hbm.at[t - 2, :, pl.ds(bt0, _BPW // 128)],
                        osem).wait()

            gather_t(tl, stg)
            pltpu.async_copy(
                stg, out_hbm.at[t, :, pl.ds(bt0, _BPW // 128)], osem)
        return carry

    lax.fori_loop(0, n_t // _TBLK, blk_body, 0)

    for t in (n_t - 2, n_t - 1):
        pltpu.make_async_copy(
            stgs[t % 2], out_hbm.at[t, :, pl.ds(bt0, _BPW // 128)],
            osem).wait()


def kernel(x, W):
    bsz, tsz = x.shape
    n_emb, emb = W.shape

    tw = pl.pallas_call(
        _tanh_table_body,
        out_shape=jax.ShapeDtypeStruct((n_emb, emb), jnp.float32),
    )(W)

    xt = jnp.transpose(x)

    mesh = plsc.VectorSubcoreMesh(
        core_axis_name="c", subcore_axis_name="s",
        num_cores=_NC, num_subcores=_NS)

    sc = functools.partial(
        pl.kernel,
        mesh=mesh,
        out_type=jax.ShapeDtypeStruct(
            (tsz, emb // 8, bsz // 128, 8, 128), jnp.float32),
        scratch_types=[
            pltpu.VMEM((n_emb, emb), jnp.float32),
            pltpu.VMEM((n_emb * _STRIDE,), jnp.float32),
            pltpu.VMEM((_TBLK, _BPW), jnp.float32),
            pltpu.VMEM((emb // 8, _BPW // 128, 8, 128), jnp.float32),
            pltpu.VMEM((emb // 8, _BPW // 128, 8, 128), jnp.float32),
            pltpu.SemaphoreType.DMA,
            pltpu.SemaphoreType.DMA,
        ],
        compiler_params=pltpu.CompilerParams(
            use_tc_tiling_on_sc=False, needs_layout_passes=False),
    )(_sc_body)

    out5 = sc(xt, tw)
    out = out5.transpose(2, 4, 0, 1, 3).reshape(bsz, tsz, emb)
    return out


# speedup vs baseline: 711.5639x; 2.5901x over previous
import functools

import jax
import jax.numpy as jnp
from jax import lax
from jax.experimental import pallas as pl
from jax.experimental.pallas import tpu as pltpu
from jax.experimental.pallas import tpu_sc as plsc

_N_BINS = 256
_EMB = 32
_NC = 2
_NS = 16
_NW = _NC * _NS
_LANES = 16

_TBLK = 8
_BPW = 512
_KV = _BPW // _LANES
_STRIDE = 33


def _tanh_table_body(w_ref, o_ref):
    o_ref[...] = jnp.tanh(w_ref[...])


def _sc_body(xt_hbm, tw_hbm, out_hbm, tabv, tab33, xblk, stg0, stg1,
             xsem, osem):
    n_t = xt_hbm.shape[0]

    wid = lax.axis_index("s") * _NC + lax.axis_index("c")
    b0 = wid * _BPW
    bt0 = wid * (_BPW // 128)

    step = jnp.float32(1.0) / jnp.float32(254.0)
    stgs = (stg0, stg1)

    pltpu.sync_copy(tw_hbm, tabv)

    def restride(r, carry):
        tab33[pl.ds(r * _STRIDE, _LANES)] = tabv[r, pl.ds(0, _LANES)]
        tab33[pl.ds(r * _STRIDE + _LANES, _LANES)] = tabv[r, pl.ds(_LANES, _LANES)]
        return carry

    lax.fori_loop(0, _N_BINS, restride, 0)

    def gather_t(tl, stg):
        @plsc.parallel_loop(0, _KV, 1, unroll=2)
        def k_body(k):
            xx = xblk[tl, pl.ds(k * _LANES, _LANES)]
            tt = xx * 254.0 + 0.5
            i0 = tt.astype(jnp.int32)
            i0 = jnp.minimum(jnp.maximum(i0, 0), 255)
            f = i0.astype(jnp.float32)
            hi = f * step
            lo = (f - 1.0) * step
            idx = (i0
                   + jnp.where(xx > hi, 1, 0)
                   - jnp.where(xx <= lo, 1, 0))
            idx33 = idx * _STRIDE
            kb = k // 8
            bl = (k % 8) * _LANES
            for d in range(_EMB):
                g = plsc.load_gather(tab33, [idx33 + d])
                stg[d // 8, kb, d % 8, pl.ds(bl, _LANES)] = g

    def blk_body(s, carry):
        t0 = s * _TBLK
        pltpu.sync_copy(
            xt_hbm.at[pl.ds(t0, _TBLK), pl.ds(b0, _BPW)], xblk)

        for tl in range(_TBLK):
            t = t0 + tl
            stg = stgs[tl % 2]

            if tl >= 2:
                pltpu.make_async_copy(
                    stg, out_hbm.at[t - 2, :, pl.ds(bt0, _BPW // 128)],
                    osem).wait()
            else:
                @pl.when(s > 0)
                def _():
                    pltpu.make_async_copy(
                        stg, out_hbm.at[t - 2, :, pl.ds(bt0, _BPW // 128)],
                        osem).wait()

            gather_t(tl, stg)
            pltpu.async_copy(
                stg, out_hbm.at[t, :, pl.ds(bt0, _BPW // 128)], osem)
        return carry

    lax.fori_loop(0, n_t // _TBLK, blk_body, 0)

    for t in (n_t - 2, n_t - 1):
        pltpu.make_async_copy(
            stgs[t % 2], out_hbm.at[t, :, pl.ds(bt0, _BPW // 128)],
            osem).wait()


def kernel(x, W):
    bsz, tsz = x.shape
    n_emb, emb = W.shape

    tw = pl.pallas_call(
        _tanh_table_body,
        out_shape=jax.ShapeDtypeStruct((n_emb, emb), jnp.float32),
    )(W)

    xt = jnp.transpose(x)

    mesh = plsc.VectorSubcoreMesh(
        core_axis_name="c", subcore_axis_name="s",
        num_cores=_NC, num_subcores=_NS)

    sc = functools.partial(
        pl.kernel,
        mesh=mesh,
        out_type=jax.ShapeDtypeStruct(
            (tsz, emb // 8, bsz // 128, 8, 128), jnp.float32),
        scratch_types=[
            pltpu.VMEM((n_emb, emb), jnp.float32),
            pltpu.VMEM((n_emb * _STRIDE,), jnp.float32),
            pltpu.VMEM((_TBLK, _BPW), jnp.float32),
            pltpu.VMEM((emb // 8, _BPW // 128, 8, 128), jnp.float32),
            pltpu.VMEM((emb // 8, _BPW // 128, 8, 128), jnp.float32),
            pltpu.SemaphoreType.DMA,
            pltpu.SemaphoreType.DMA,
        ],
        compiler_params=pltpu.CompilerParams(
            use_tc_tiling_on_sc=False, needs_layout_passes=False),
    )(_sc_body)

    out5 = sc(xt, tw)
    out = out5.transpose(2, 4, 0, 1, 3).reshape(bsz, tsz, emb)
    return out
